# Initial kernel scaffold; baseline (speedup 1.0000x reference)
import jax
import jax.numpy as jnp
from jax.experimental import pallas as pl


def kernel(x, edge_index0, edge_index1, edge_index2, params):
    raise NotImplementedError("write your pallas kernel here")


# speedup vs baseline: 8.3950x; 8.3950x over previous
import functools

import jax
import jax.numpy as jnp
from jax.experimental import pallas as pl

_EPS_BN = 1e-5
_N_HEADS = 16
_HEAD_DIM = 16


def _mlp_body(x_ref, w1_ref, b1_ref, g_ref, beta_ref, w2_ref, b2_ref, o_ref,
              *, skip, pre_lrelu):
    x = x_ref[...]
    if pre_lrelu:
        x = jnp.where(x >= 0, x, 0.01 * x)
    h = jnp.dot(x, w1_ref[...], preferred_element_type=jnp.float32) + b1_ref[...]
    mu = jnp.mean(h, axis=0, keepdims=True)
    var = jnp.mean((h - mu) * (h - mu), axis=0, keepdims=True)
    h = (h - mu) * jax.lax.rsqrt(var + _EPS_BN) * g_ref[...] + beta_ref[...]
    h = jnp.maximum(h, 0.0)
    out = jnp.dot(h, w2_ref[...], preferred_element_type=jnp.float32) + b2_ref[...]
    if skip:
        out = out + h
    o_ref[...] = out


def _mlp_fwd(x, p, skip, pre_lrelu=False):
    n, dout = x.shape[0], p['w2'].shape[1]
    body = functools.partial(_mlp_body, skip=skip, pre_lrelu=pre_lrelu)
    return pl.pallas_call(
        body,
        out_shape=jax.ShapeDtypeStruct((n, dout), jnp.float32),
    )(x, p['w1'], p['b1'].reshape(1, -1), p['g'].reshape(1, -1),
      p['beta'].reshape(1, -1), p['w2'], p['b2'].reshape(1, -1))


def _gat_dense_body(h_ref, w_ref, al_ref, ar_ref, feat_ref, el_ref, er_ref):
    feat = jnp.dot(h_ref[...], w_ref[...], preferred_element_type=jnp.float32)
    feat_ref[...] = feat
    el_ref[...] = jnp.dot(feat, al_ref[...], preferred_element_type=jnp.float32)
    er_ref[...] = jnp.dot(feat, ar_ref[...], preferred_element_type=jnp.float32)


def _gat_dense(h, w, al_mat, ar_mat):
    n = h.shape[0]
    d = w.shape[1]
    return pl.pallas_call(
        _gat_dense_body,
        out_shape=(
            jax.ShapeDtypeStruct((n, d), jnp.float32),
            jax.ShapeDtypeStruct((n, _N_HEADS), jnp.float32),
            jax.ShapeDtypeStruct((n, _N_HEADS), jnp.float32),
        ),
    )(h, w, al_mat, ar_mat)


def _block_diag_att(a):
    hd = _N_HEADS * _HEAD_DIM
    rows = jnp.arange(hd)
    return jnp.zeros((hd, _N_HEADS), jnp.float32).at[rows, rows // _HEAD_DIM].set(
        a.reshape(-1))


def _gat_conv(h, src, dst, p, n_nodes):
    feat, el, er = _gat_dense(h, p['w'], _block_diag_att(p['al']),
                              _block_diag_att(p['ar']))
    e = el[src] + er[dst]
    e = jnp.where(e >= 0, e, 0.2 * e)
    emax = jax.ops.segment_max(e, dst, num_segments=n_nodes)
    emax = jnp.where(jnp.isfinite(emax), emax, 0.0)
    ee = jnp.exp(e - emax[dst])
    denom = jax.ops.segment_sum(ee, dst, num_segments=n_nodes)
    alpha = ee / jnp.maximum(denom[dst], 1e-16)
    msg = feat[src] * jnp.repeat(alpha, _HEAD_DIM, axis=1)
    out = jax.ops.segment_sum(msg, dst, num_segments=n_nodes)
    return out + p['bias'].reshape(1, -1)


def kernel(x, edge_index0, edge_index1, edge_index2, params):
    n_nodes = x.shape[0]
    eis = [edge_index0, edge_index1, edge_index2]
    h = _mlp_fwd(x, params['embed'], skip=True)
    n_layers = len(params['mlp'])
    for l in range(n_layers):
        agg = jnp.zeros((n_nodes, _N_HEADS * _HEAD_DIM), jnp.float32)
        for r in range(len(eis)):
            src, dst = eis[r][0], eis[r][1]
            agg = agg + _gat_conv(h, src, dst, params['gat'][l][r], n_nodes)
        hn = _mlp_fwd(agg, params['mlp'][l], skip=False, pre_lrelu=True)
        h = hn + h
    return _mlp_fwd(h, params['dec'], skip=False)
